# baseline (speedup 1.0000x reference)
import functools

import jax
import jax.numpy as jnp
from jax import lax
from jax.experimental import pallas as pl
from jax.experimental.pallas import tpu as pltpu
from jax.experimental.pallas import tpu_sc as plsc

NN = 10000
EE = 320000
DD = 128
HH = 64
RR = 32
BB = 64

NC = 2
NS = 16
LL = 16
NW = NC * NS

KK = 80
NB = 5
ET = EE // NS
NGR = ET // (NB * KK)
NRR = NN * RR
ZB = 2000
ZR = 40
NCH = NN // ZR
CPT = -(-NCH // NS)

_mesh = plsc.VectorSubcoreMesh(core_axis_name="c", subcore_axis_name="s",
                               num_cores=NC, num_subcores=NS)
_sc_params = pltpu.CompilerParams(use_tc_tiling_on_sc=False)


def _zero_rows(zrow):
    for r in range(ZR):
        for j in range(HH // LL):
            zrow[r, pl.ds(j * LL, LL)] = jnp.zeros((LL,), jnp.float32)


def _zero_agg(agg, zrow, s):
    def zc(jj, _):
        j = jj * NS + s

        @pl.when(j < NCH)
        def _():
            pltpu.sync_copy(zrow, agg.at[pl.ds(j * ZR, ZR)])
        return 0
    lax.fori_loop(0, CPT, zc, 0)


def _copy_out(agg, aggp_hbm, c, s):
    def oc(jj, _):
        j = jj * NS + s

        @pl.when(j < NCH)
        def _():
            pltpu.sync_copy(agg.at[pl.ds(j * ZR, ZR)],
                            aggp_hbm.at[pl.ds(c * NN + j * ZR, ZR)])
        return 0
    lax.fori_loop(0, CPT, oc, 0)


def _scale_rows(rows_vs, norm_vs, b):
    for j16 in range(KK // LL):
        nv16 = norm_vs[b, pl.ds(j16 * LL, LL)]
        for l in range(LL):
            e = j16 * LL + l
            nv = nv16[l]
            for j in range(HH // LL):
                sl = pl.ds(j * LL, LL)
                rows_vs[b, e, sl] = rows_vs[b, e, sl] * nv


def _sc_l1_body(srcb, dstb, etb, xrel, normb, aggp_hbm,
                counts, agg, src_vs, dst_vs, et_vs, comb_vs, midx_vs,
                ones_v, cnt_vs, norm_vs, rows_vs, zbuf, zrow, *sems):
    lsems = sems[0:NB]
    gsems = sems[NB:2 * NB]
    csems = sems[2 * NB:3 * NB]
    ssems = sems[3 * NB:4 * NB]
    asems = sems[4 * NB:5 * NB]
    c = lax.axis_index("c")
    s = lax.axis_index("s")

    for j in range(KK // LL):
        ones_v[pl.ds(j * LL, LL)] = jnp.ones((LL,), jnp.float32)

    def zfill(i, _):
        zbuf[pl.ds(i * LL, LL)] = jnp.zeros((LL,), jnp.float32)
        return 0
    lax.fori_loop(0, ZB // LL, zfill, 0)

    def zcopy(j, _):
        pltpu.sync_copy(zbuf, counts.at[pl.ds(s * (NRR // NS) + j * ZB, ZB)])
        return 0
    lax.fori_loop(0, (NRR // NS) // ZB, zcopy, 0)
    _zero_rows(zrow)
    _zero_agg(agg, zrow, s)
    plsc.subcore_barrier()

    def p1(g, _):
        base0 = s * ET + g * (NB * KK)
        ldescs = []
        for b in range(NB):
            base = base0 + b * KK
            d1 = pltpu.async_copy(dstb.at[c, pl.ds(base, KK)],
                                  dst_vs.at[b], lsems[b])
            d2 = pltpu.async_copy(etb.at[c, pl.ds(base, KK)],
                                  et_vs.at[b], lsems[b])
            ldescs.append((d1, d2))
        adescs = []
        for b in range(NB):
            for d in ldescs[b]:
                d.wait()
            for j in range(KK // LL):
                sl = pl.ds(j * LL, LL)
                comb_vs[b, sl] = dst_vs[b, sl] * RR + et_vs[b, sl]
            adescs.append(pltpu.async_copy(ones_v, counts.at[comb_vs.at[b]],
                                           gsems[b], add=True))
        for b in range(NB):
            adescs[b].wait()
        return 0
    lax.fori_loop(0, NGR, p1, 0)
    plsc.subcore_barrier()

    def p2(g, _):
        base0 = s * ET + g * (NB * KK)
        ldescs = []
        for b in range(NB):
            base = base0 + b * KK
            d1 = pltpu.async_copy(srcb.at[c, pl.ds(base, KK)],
                                  src_vs.at[b], lsems[b])
            d2 = pltpu.async_copy(dstb.at[c, pl.ds(base, KK)],
                                  dst_vs.at[b], lsems[b])
            d3 = pltpu.async_copy(etb.at[c, pl.ds(base, KK)],
                                  et_vs.at[b], lsems[b])
            ldescs.append((d1, d2, d3))
        cdescs = []
        for b in range(NB):
            for d in ldescs[b]:
                d.wait()
            for j in range(KK // LL):
                sl = pl.ds(j * LL, LL)
                comb_vs[b, sl] = dst_vs[b, sl] * RR + et_vs[b, sl]
                midx_vs[b, sl] = (et_vs[b, sl] * NN + src_vs[b, sl]
                                  + c * NRR)
            g1 = pltpu.async_copy(counts.at[comb_vs.at[b]],
                                  cnt_vs.at[b], gsems[b])
            g2 = pltpu.async_copy(xrel.at[midx_vs.at[b]],
                                  rows_vs.at[b], csems[b])
            cdescs.append((g1, g2))
        sdescs = []
        for b in range(NB):
            base = base0 + b * KK
            cdescs[b][0].wait()
            for j in range(KK // LL):
                sl = pl.ds(j * LL, LL)
                norm_vs[b, sl] = 1.0 / jnp.maximum(cnt_vs[b, sl], 1.0)
            sd = pltpu.async_copy(norm_vs.at[b],
                                  normb.at[c, pl.ds(base, KK)], ssems[b])
            cdescs[b][1].wait()
            _scale_rows(rows_vs, norm_vs, b)
            ad = pltpu.async_copy(rows_vs.at[b], agg.at[dst_vs.at[b]],
                                  asems[b], add=True)
            sdescs.append((sd, ad))
        for b in range(NB):
            for d in sdescs[b]:
                d.wait()
        return 0
    lax.fori_loop(0, NGR, p2, 0)
    plsc.subcore_barrier()

    _copy_out(agg, aggp_hbm, c, s)


@functools.partial(
    pl.kernel,
    out_type=[jax.ShapeDtypeStruct((NC, EE), jnp.float32),
              jax.ShapeDtypeStruct((NC * NN, HH), jnp.float32)],
    mesh=_mesh,
    compiler_params=_sc_params,
    scratch_types=[
        pltpu.VMEM_SHARED((NRR,), jnp.float32),
        pltpu.VMEM_SHARED((NN, HH), jnp.float32),
        pltpu.VMEM((NB, KK), jnp.int32),
        pltpu.VMEM((NB, KK), jnp.int32),
        pltpu.VMEM((NB, KK), jnp.int32),
        pltpu.VMEM((NB, KK), jnp.int32),
        pltpu.VMEM((NB, KK), jnp.int32),
        pltpu.VMEM((KK,), jnp.float32),
        pltpu.VMEM((NB, KK), jnp.float32),
        pltpu.VMEM((NB, KK), jnp.float32),
        pltpu.VMEM((NB, KK, HH), jnp.float32),
        pltpu.VMEM((ZB,), jnp.float32),
        pltpu.VMEM((ZR, HH), jnp.float32),
    ] + [pltpu.SemaphoreType.DMA] * (5 * NB),
)
def _sc_l1(srcb, dstb, etb, xrel, *rest):
    _sc_l1_body(srcb, dstb, etb, xrel, *rest)


def _sc_l2_body(srcb, dstb, etb, normb, xrel, aggp_hbm,
                agg, src_vs, dst_vs, et_vs, midx_vs, norm_vs, rows_vs,
                zrow, *sems):
    lsems = sems[0:NB]
    gsems = sems[NB:2 * NB]
    ssems = sems[2 * NB:3 * NB]
    c = lax.axis_index("c")
    s = lax.axis_index("s")

    _zero_rows(zrow)
    _zero_agg(agg, zrow, s)
    plsc.subcore_barrier()

    def group(g, _):
        base0 = s * ET + g * (NB * KK)
        ldescs = []
        for b in range(NB):
            base = base0 + b * KK
            d1 = pltpu.async_copy(srcb.at[c, pl.ds(base, KK)],
                                  src_vs.at[b], lsems[b])
            d2 = pltpu.async_copy(dstb.at[c, pl.ds(base, KK)],
                                  dst_vs.at[b], lsems[b])
            d3 = pltpu.async_copy(etb.at[c, pl.ds(base, KK)],
                                  et_vs.at[b], lsems[b])
            d4 = pltpu.async_copy(normb.at[c, pl.ds(base, KK)],
                                  norm_vs.at[b], lsems[b])
            ldescs.append((d1, d2, d3, d4))
        gdescs = []
        for b in range(NB):
            for d in ldescs[b]:
                d.wait()
            for j in range(KK // LL):
                sl = pl.ds(j * LL, LL)
                midx_vs[b, sl] = (et_vs[b, sl] * NN + src_vs[b, sl]
                                  + c * NRR)
            gdescs.append(pltpu.async_copy(xrel.at[midx_vs.at[b]],
                                           rows_vs.at[b], gsems[b]))
        sdescs = []
        for b in range(NB):
            gdescs[b].wait()
            _scale_rows(rows_vs, norm_vs, b)
            sdescs.append(pltpu.async_copy(rows_vs.at[b],
                                           agg.at[dst_vs.at[b]],
                                           ssems[b], add=True))
        for d in sdescs:
            d.wait()
        return 0
    lax.fori_loop(0, NGR, group, 0)
    plsc.subcore_barrier()

    _copy_out(agg, aggp_hbm, c, s)


@functools.partial(
    pl.kernel,
    out_type=jax.ShapeDtypeStruct((NC * NN, HH), jnp.float32),
    mesh=_mesh,
    compiler_params=_sc_params,
    scratch_types=[
        pltpu.VMEM_SHARED((NN, HH), jnp.float32),
        pltpu.VMEM((NB, KK), jnp.int32),
        pltpu.VMEM((NB, KK), jnp.int32),
        pltpu.VMEM((NB, KK), jnp.int32),
        pltpu.VMEM((NB, KK), jnp.int32),
        pltpu.VMEM((NB, KK), jnp.float32),
        pltpu.VMEM((NB, KK, HH), jnp.float32),
        pltpu.VMEM((ZR, HH), jnp.float32),
    ] + [pltpu.SemaphoreType.DMA] * (3 * NB),
)
def _sc_l2(srcb, dstb, etb, normb, xrel, *rest):
    _sc_l2_body(srcb, dstb, etb, normb, xrel, *rest)


MM = 400
GG = NN // MM


def _mm1_body(x_ref, w_ref, rt_ref, xrel_ref, rto_ref):
    q = pl.program_id(2)
    x = x_ref[0]
    xrel_ref[...] = jnp.dot(x, w_ref[0, 0],
                            preferred_element_type=jnp.float32)

    @pl.when(q == 0)
    def _():
        rto_ref[...] = jnp.dot(x, rt_ref[0],
                               preferred_element_type=jnp.float32)


def _tc_mm1(xb, wb, rtb):
    din = wb.shape[2]
    return pl.pallas_call(
        _mm1_body,
        grid=(NC, GG, RR),
        in_specs=[pl.BlockSpec((1, MM, din), lambda e, i, q: (e, i, 0)),
                  pl.BlockSpec((1, 1, din, HH), lambda e, i, q: (e, q, 0, 0)),
                  pl.BlockSpec((1, din, HH), lambda e, i, q: (e, 0, 0))],
        out_specs=[pl.BlockSpec((MM, HH),
                                lambda e, i, q: (e * (RR * GG) + q * GG + i,
                                                 0)),
                   pl.BlockSpec((MM, HH), lambda e, i, q: (e * GG + i, 0))],
        out_shape=[jax.ShapeDtypeStruct((NC * NRR, HH), jnp.float32),
                   jax.ShapeDtypeStruct((NC * NN, HH), jnp.float32)],
    )(xb, wb, rtb)


def _mm2_body(agg_ref, rt1_ref, b1_ref, w_ref, rt2_ref, xrel_ref, rto_ref):
    q = pl.program_id(2)
    h = jnp.maximum(agg_ref[...] + rt1_ref[...] + b1_ref[0], 0.0)
    xrel_ref[...] = jnp.dot(h, w_ref[0, 0],
                            preferred_element_type=jnp.float32)

    @pl.when(q == 0)
    def _():
        rto_ref[...] = jnp.dot(h, rt2_ref[0],
                               preferred_element_type=jnp.float32)


def _tc_mm2(agg, rt1, b1b, wb, rtb):
    din = wb.shape[2]
    return pl.pallas_call(
        _mm2_body,
        grid=(NC, GG, RR),
        in_specs=[pl.BlockSpec((MM, HH), lambda e, i, q: (e * GG + i, 0)),
                  pl.BlockSpec((MM, HH), lambda e, i, q: (e * GG + i, 0)),
                  pl.BlockSpec((1, 1, HH), lambda e, i, q: (e, 0, 0)),
                  pl.BlockSpec((1, 1, din, HH), lambda e, i, q: (e, q, 0, 0)),
                  pl.BlockSpec((1, din, HH), lambda e, i, q: (e, 0, 0))],
        out_specs=[pl.BlockSpec((MM, HH),
                                lambda e, i, q: (e * (RR * GG) + q * GG + i,
                                                 0)),
                   pl.BlockSpec((MM, HH), lambda e, i, q: (e * GG + i, 0))],
        out_shape=[jax.ShapeDtypeStruct((NC * NRR, HH), jnp.float32),
                   jax.ShapeDtypeStruct((NC * NN, HH), jnp.float32)],
    )(agg, rt1, b1b, wb, rtb)


def _pool_body(agg_ref, rt2_ref, b2_ref, batch_ref, psum_ref, pcnt_ref):
    i = pl.program_id(1)
    h = jnp.maximum(agg_ref[...] + rt2_ref[...] + b2_ref[0], 0.0)
    bt = batch_ref[0, 0, 0, :]
    oh = (bt[None, :] == lax.broadcasted_iota(jnp.int32, (BB, MM), 0)
          ).astype(jnp.float32)
    ps = jnp.dot(oh, h, preferred_element_type=jnp.float32)
    pc = jnp.sum(oh, axis=1)[None, :]

    @pl.when(i == 0)
    def _():
        psum_ref[...] = jnp.zeros_like(psum_ref)
        pcnt_ref[...] = jnp.zeros_like(pcnt_ref)
    psum_ref[0] += ps
    pcnt_ref[0] += pc


def _tc_pool(agg, rt2, b2b, batchb):
    return pl.pallas_call(
        _pool_body,
        grid=(NC, GG),
        in_specs=[pl.BlockSpec((MM, HH), lambda e, i: (e * GG + i, 0)),
                  pl.BlockSpec((MM, HH), lambda e, i: (e * GG + i, 0)),
                  pl.BlockSpec((1, 1, HH), lambda e, i: (e, 0, 0)),
                  pl.BlockSpec((1, 1, 1, MM), lambda e, i: (e, i, 0, 0))],
        out_specs=[pl.BlockSpec((1, BB, HH), lambda e, i: (e, 0, 0)),
                   pl.BlockSpec((1, 1, BB), lambda e, i: (e, 0, 0))],
        out_shape=[jax.ShapeDtypeStruct((NC, BB, HH), jnp.float32),
                   jax.ShapeDtypeStruct((NC, 1, BB), jnp.float32)],
    )(agg, rt2, b2b, batchb)


def _tail_body(ps_ref, pc_ref, d_ref, w1a_ref, w1b_ref,
               w1c_ref, b1_ref, w2r_ref, b2_ref, out_ref):
    se = ps_ref[0] / jnp.maximum(pc_ref[0], 1.0)
    ge = ps_ref[1] / jnp.maximum(pc_ref[1], 1.0)
    d = d_ref[...]
    dm = jnp.mean(d)
    sd = jnp.sqrt(jnp.mean((d - dm) ** 2))
    dn = (d - dm) / (sd + 1e-6)
    z = (jnp.dot(se, w1a_ref[...], preferred_element_type=jnp.float32)
         + jnp.dot(ge, w1b_ref[...], preferred_element_type=jnp.float32)
         + dn * w1c_ref[...] + b1_ref[...])
    hh = jnp.maximum(z, 0.0)
    out_ref[...] = jnp.sum(hh * w2r_ref[...], axis=1, keepdims=True) \
        + b2_ref[...]


def _tc_tail(ps, pc, d, w1a, w1b, w1c, b1, w2r, b2):
    return pl.pallas_call(
        _tail_body,
        out_shape=jax.ShapeDtypeStruct((BB, 1), jnp.float32),
    )(ps, pc, d, w1a, w1b, w1c, b1, w2r, b2)


def kernel(state_x, state_edge_index, state_edge_type, state_batch,
           goal_x, goal_edge_index, goal_edge_type, goal_batch, depth,
           s1_W, s1_root, s1_b, s2_W, s2_root, s2_b,
           g1_W, g1_root, g1_b, g2_W, g2_root, g2_b,
           reg_W1, reg_b1, reg_W2, reg_b2):
    srcb = jnp.stack([state_edge_index[0], goal_edge_index[0]])
    dstb = jnp.stack([state_edge_index[1], goal_edge_index[1]])
    etb = jnp.stack([state_edge_type, goal_edge_type])
    xb = jnp.stack([state_x, goal_x])
    batchb = jnp.stack([state_batch, goal_batch]).reshape(NC, GG, 1, MM)

    xrel1, rt1 = _tc_mm1(xb, jnp.stack([s1_W, g1_W]),
                         jnp.stack([s1_root, g1_root]))
    normb, agg1 = _sc_l1(srcb, dstb, etb, xrel1)
    xrel2, rt2 = _tc_mm2(agg1, rt1,
                         jnp.stack([s1_b, g1_b]).reshape(NC, 1, HH),
                         jnp.stack([s2_W, g2_W]),
                         jnp.stack([s2_root, g2_root]))
    agg2 = _sc_l2(srcb, dstb, etb, normb, xrel2)
    ps, pc = _tc_pool(agg2, rt2,
                      jnp.stack([s2_b, g2_b]).reshape(NC, 1, HH), batchb)
    pred = _tc_tail(ps, jnp.swapaxes(pc, 1, 2),
                    depth.reshape(BB, 1),
                    reg_W1[:HH], reg_W1[HH:2 * HH], reg_W1[2 * HH:],
                    reg_b1.reshape(1, HH), reg_W2.reshape(1, HH),
                    reg_b2.reshape(1, 1))
    return pred.reshape(BB)


# speedup vs baseline: 2.9166x; 2.9166x over previous
import functools

import jax
import jax.numpy as jnp
from jax import lax
from jax.experimental import pallas as pl
from jax.experimental.pallas import tpu as pltpu
from jax.experimental.pallas import tpu_sc as plsc

NN = 10000
EE = 320000
DD = 128
HH = 64
RR = 32
BB = 64

NC = 2
NS = 16
LL = 16
NW = NC * NS

KK = 80
NB = 5
ET = EE // NS
NGR = ET // (NB * KK)
NRR = NN * RR
ZB = 2000
ZR = 40
NCH = NN // ZR
CPT = -(-NCH // NS)

_mesh = plsc.VectorSubcoreMesh(core_axis_name="c", subcore_axis_name="s",
                               num_cores=NC, num_subcores=NS)
_sc_params = pltpu.CompilerParams(use_tc_tiling_on_sc=False)


def _zero_rows(zrow):
    for r in range(ZR):
        for j in range(HH // LL):
            zrow[r, pl.ds(j * LL, LL)] = jnp.zeros((LL,), jnp.float32)


def _zero_agg(agg, zrow, s):
    def zc(jj, _):
        j = jj * NS + s

        @pl.when(j < NCH)
        def _():
            pltpu.sync_copy(zrow, agg.at[pl.ds(j * ZR, ZR)])
        return 0
    lax.fori_loop(0, CPT, zc, 0)


def _copy_out(agg, aggp_hbm, c, s):
    def oc(jj, _):
        j = jj * NS + s

        @pl.when(j < NCH)
        def _():
            pltpu.sync_copy(agg.at[pl.ds(j * ZR, ZR)],
                            aggp_hbm.at[pl.ds(c * NN + j * ZR, ZR)])
        return 0
    lax.fori_loop(0, CPT, oc, 0)


def _scale_rows(rows_vs, norm_vs, b):
    for j16 in range(KK // LL):
        nv16 = norm_vs[b, pl.ds(j16 * LL, LL)]
        for l in range(LL):
            e = j16 * LL + l
            nv = nv16[l]
            for j in range(HH // LL):
                sl = pl.ds(j * LL, LL)
                rows_vs[b, e, sl] = rows_vs[b, e, sl] * nv


def _sc_l1_body(srcb, dstb, etb, xrel, normb, aggp_hbm,
                counts, agg, src_vs, dst_vs, et_vs, comb_vs, midx_vs,
                ones_v, cnt_vs, norm_vs, rows_vs, zbuf, zrow, *sems):
    lsems = sems[0:NB]
    gsems = sems[NB:2 * NB]
    csems = sems[2 * NB:3 * NB]
    ssems = sems[3 * NB:4 * NB]
    asems = sems[4 * NB:5 * NB]
    c = lax.axis_index("c")
    s = lax.axis_index("s")

    for j in range(KK // LL):
        ones_v[pl.ds(j * LL, LL)] = jnp.ones((LL,), jnp.float32)

    def zfill(i, _):
        zbuf[pl.ds(i * LL, LL)] = jnp.zeros((LL,), jnp.float32)
        return 0
    lax.fori_loop(0, ZB // LL, zfill, 0)

    def zcopy(j, _):
        pltpu.sync_copy(zbuf, counts.at[pl.ds(s * (NRR // NS) + j * ZB, ZB)])
        return 0
    lax.fori_loop(0, (NRR // NS) // ZB, zcopy, 0)
    _zero_rows(zrow)
    _zero_agg(agg, zrow, s)
    plsc.subcore_barrier()

    def p1(g, _):
        base0 = s * ET + g * (NB * KK)
        ldescs = []
        for b in range(NB):
            base = base0 + b * KK
            d1 = pltpu.async_copy(dstb.at[c, pl.ds(base, KK)],
                                  dst_vs.at[b], lsems[b])
            d2 = pltpu.async_copy(etb.at[c, pl.ds(base, KK)],
                                  et_vs.at[b], lsems[b])
            ldescs.append((d1, d2))
        adescs = []
        for b in range(NB):
            for d in ldescs[b]:
                d.wait()
            for j in range(KK // LL):
                sl = pl.ds(j * LL, LL)
                comb_vs[b, sl] = dst_vs[b, sl] * RR + et_vs[b, sl]
            adescs.append(pltpu.async_copy(ones_v, counts.at[comb_vs.at[b]],
                                           gsems[b], add=True))
        for b in range(NB):
            adescs[b].wait()
        return 0
    lax.fori_loop(0, NGR, p1, 0)
    plsc.subcore_barrier()

    def p2(g, _):
        base0 = s * ET + g * (NB * KK)
        ldescs = []
        for b in range(NB):
            base = base0 + b * KK
            d1 = pltpu.async_copy(srcb.at[c, pl.ds(base, KK)],
                                  src_vs.at[b], lsems[b])
            d2 = pltpu.async_copy(dstb.at[c, pl.ds(base, KK)],
                                  dst_vs.at[b], lsems[b])
            d3 = pltpu.async_copy(etb.at[c, pl.ds(base, KK)],
                                  et_vs.at[b], lsems[b])
            ldescs.append((d1, d2, d3))
        cdescs = []
        for b in range(NB):
            for d in ldescs[b]:
                d.wait()
            for j in range(KK // LL):
                sl = pl.ds(j * LL, LL)
                comb_vs[b, sl] = dst_vs[b, sl] * RR + et_vs[b, sl]
                midx_vs[b, sl] = (src_vs[b, sl] * RR + et_vs[b, sl]
                                  + c * NRR)
            g1 = pltpu.async_copy(counts.at[comb_vs.at[b]],
                                  cnt_vs.at[b], gsems[b])
            g2 = pltpu.async_copy(xrel.at[midx_vs.at[b]],
                                  rows_vs.at[b], csems[b])
            cdescs.append((g1, g2))
        sdescs = []
        for b in range(NB):
            base = base0 + b * KK
            cdescs[b][0].wait()
            for j in range(KK // LL):
                sl = pl.ds(j * LL, LL)
                norm_vs[b, sl] = 1.0 / jnp.maximum(cnt_vs[b, sl], 1.0)
            sd = pltpu.async_copy(norm_vs.at[b],
                                  normb.at[c, pl.ds(base, KK)], ssems[b])
            cdescs[b][1].wait()
            _scale_rows(rows_vs, norm_vs, b)
            ad = pltpu.async_copy(rows_vs.at[b], agg.at[dst_vs.at[b]],
                                  asems[b], add=True)
            sdescs.append((sd, ad))
        for b in range(NB):
            for d in sdescs[b]:
                d.wait()
        return 0
    lax.fori_loop(0, NGR, p2, 0)
    plsc.subcore_barrier()

    _copy_out(agg, aggp_hbm, c, s)


@functools.partial(
    pl.kernel,
    out_type=[jax.ShapeDtypeStruct((NC, EE), jnp.float32),
              jax.ShapeDtypeStruct((NC * NN, HH), jnp.float32)],
    mesh=_mesh,
    compiler_params=_sc_params,
    scratch_types=[
        pltpu.VMEM_SHARED((NRR,), jnp.float32),
        pltpu.VMEM_SHARED((NN, HH), jnp.float32),
        pltpu.VMEM((NB, KK), jnp.int32),
        pltpu.VMEM((NB, KK), jnp.int32),
        pltpu.VMEM((NB, KK), jnp.int32),
        pltpu.VMEM((NB, KK), jnp.int32),
        pltpu.VMEM((NB, KK), jnp.int32),
        pltpu.VMEM((KK,), jnp.float32),
        pltpu.VMEM((NB, KK), jnp.float32),
        pltpu.VMEM((NB, KK), jnp.float32),
        pltpu.VMEM((NB, KK, HH), jnp.float32),
        pltpu.VMEM((ZB,), jnp.float32),
        pltpu.VMEM((ZR, HH), jnp.float32),
    ] + [pltpu.SemaphoreType.DMA] * (5 * NB),
)
def _sc_l1(srcb, dstb, etb, xrel, *rest):
    _sc_l1_body(srcb, dstb, etb, xrel, *rest)


def _sc_l2_body(srcb, dstb, etb, normb, xrel, aggp_hbm,
                agg, src_vs, dst_vs, et_vs, midx_vs, norm_vs, rows_vs,
                zrow, *sems):
    lsems = sems[0:NB]
    gsems = sems[NB:2 * NB]
    ssems = sems[2 * NB:3 * NB]
    c = lax.axis_index("c")
    s = lax.axis_index("s")

    _zero_rows(zrow)
    _zero_agg(agg, zrow, s)
    plsc.subcore_barrier()

    def group(g, _):
        base0 = s * ET + g * (NB * KK)
        ldescs = []
        for b in range(NB):
            base = base0 + b * KK
            d1 = pltpu.async_copy(srcb.at[c, pl.ds(base, KK)],
                                  src_vs.at[b], lsems[b])
            d2 = pltpu.async_copy(dstb.at[c, pl.ds(base, KK)],
                                  dst_vs.at[b], lsems[b])
            d3 = pltpu.async_copy(etb.at[c, pl.ds(base, KK)],
                                  et_vs.at[b], lsems[b])
            d4 = pltpu.async_copy(normb.at[c, pl.ds(base, KK)],
                                  norm_vs.at[b], lsems[b])
            ldescs.append((d1, d2, d3, d4))
        gdescs = []
        for b in range(NB):
            for d in ldescs[b]:
                d.wait()
            for j in range(KK // LL):
                sl = pl.ds(j * LL, LL)
                midx_vs[b, sl] = (src_vs[b, sl] * RR + et_vs[b, sl]
                                  + c * NRR)
            gdescs.append(pltpu.async_copy(xrel.at[midx_vs.at[b]],
                                           rows_vs.at[b], gsems[b]))
        sdescs = []
        for b in range(NB):
            gdescs[b].wait()
            _scale_rows(rows_vs, norm_vs, b)
            sdescs.append(pltpu.async_copy(rows_vs.at[b],
                                           agg.at[dst_vs.at[b]],
                                           ssems[b], add=True))
        for d in sdescs:
            d.wait()
        return 0
    lax.fori_loop(0, NGR, group, 0)
    plsc.subcore_barrier()

    _copy_out(agg, aggp_hbm, c, s)


@functools.partial(
    pl.kernel,
    out_type=jax.ShapeDtypeStruct((NC * NN, HH), jnp.float32),
    mesh=_mesh,
    compiler_params=_sc_params,
    scratch_types=[
        pltpu.VMEM_SHARED((NN, HH), jnp.float32),
        pltpu.VMEM((NB, KK), jnp.int32),
        pltpu.VMEM((NB, KK), jnp.int32),
        pltpu.VMEM((NB, KK), jnp.int32),
        pltpu.VMEM((NB, KK), jnp.int32),
        pltpu.VMEM((NB, KK), jnp.float32),
        pltpu.VMEM((NB, KK, HH), jnp.float32),
        pltpu.VMEM((ZR, HH), jnp.float32),
    ] + [pltpu.SemaphoreType.DMA] * (3 * NB),
)
def _sc_l2(srcb, dstb, etb, normb, xrel, *rest):
    _sc_l2_body(srcb, dstb, etb, normb, xrel, *rest)


MM = 400
GG = NN // MM


def _mm1_body(x_ref, w_ref, rt_ref, xrel_ref, rto_ref):
    x = x_ref[0]
    xrel_ref[...] = jnp.dot(x, w_ref[0],
                            preferred_element_type=jnp.float32)
    rto_ref[...] = jnp.dot(x, rt_ref[0],
                           preferred_element_type=jnp.float32)


def _tc_mm1(xb, wb, rtb):
    din = wb.shape[1]
    return pl.pallas_call(
        _mm1_body,
        grid=(NC, GG),
        in_specs=[pl.BlockSpec((1, MM, din), lambda e, i: (e, i, 0)),
                  pl.BlockSpec((1, din, RR * HH), lambda e, i: (e, 0, 0)),
                  pl.BlockSpec((1, din, HH), lambda e, i: (e, 0, 0))],
        out_specs=[pl.BlockSpec((MM, RR * HH), lambda e, i: (e * GG + i, 0)),
                   pl.BlockSpec((MM, HH), lambda e, i: (e * GG + i, 0))],
        out_shape=[jax.ShapeDtypeStruct((NC * NN, RR * HH), jnp.float32),
                   jax.ShapeDtypeStruct((NC * NN, HH), jnp.float32)],
    )(xb, wb, rtb)


def _mm2_body(agg_ref, rt1_ref, b1_ref, w_ref, rt2_ref, xrel_ref, rto_ref):
    h = jnp.maximum(agg_ref[...] + rt1_ref[...] + b1_ref[0], 0.0)
    xrel_ref[...] = jnp.dot(h, w_ref[0],
                            preferred_element_type=jnp.float32)
    rto_ref[...] = jnp.dot(h, rt2_ref[0],
                           preferred_element_type=jnp.float32)


def _tc_mm2(agg, rt1, b1b, wb, rtb):
    din = wb.shape[1]
    return pl.pallas_call(
        _mm2_body,
        grid=(NC, GG),
        in_specs=[pl.BlockSpec((MM, HH), lambda e, i: (e * GG + i, 0)),
                  pl.BlockSpec((MM, HH), lambda e, i: (e * GG + i, 0)),
                  pl.BlockSpec((1, 1, HH), lambda e, i: (e, 0, 0)),
                  pl.BlockSpec((1, din, RR * HH), lambda e, i: (e, 0, 0)),
                  pl.BlockSpec((1, din, HH), lambda e, i: (e, 0, 0))],
        out_specs=[pl.BlockSpec((MM, RR * HH), lambda e, i: (e * GG + i, 0)),
                   pl.BlockSpec((MM, HH), lambda e, i: (e * GG + i, 0))],
        out_shape=[jax.ShapeDtypeStruct((NC * NN, RR * HH), jnp.float32),
                   jax.ShapeDtypeStruct((NC * NN, HH), jnp.float32)],
    )(agg, rt1, b1b, wb, rtb)


def _pool_body(agg_ref, rt2_ref, b2_ref, batch_ref, psum_ref, pcnt_ref):
    i = pl.program_id(1)
    h = jnp.maximum(agg_ref[...] + rt2_ref[...] + b2_ref[0], 0.0)
    bt = batch_ref[0, 0, 0, :]
    oh = (bt[None, :] == lax.broadcasted_iota(jnp.int32, (BB, MM), 0)
          ).astype(jnp.float32)
    ps = jnp.dot(oh, h, preferred_element_type=jnp.float32,
                 precision=lax.Precision.HIGHEST)
    pc = jnp.sum(oh, axis=1)[None, :]

    @pl.when(i == 0)
    def _():
        psum_ref[...] = jnp.zeros_like(psum_ref)
        pcnt_ref[...] = jnp.zeros_like(pcnt_ref)
    psum_ref[0] += ps
    pcnt_ref[0] += pc


def _tc_pool(agg, rt2, b2b, batchb):
    return pl.pallas_call(
        _pool_body,
        grid=(NC, GG),
        in_specs=[pl.BlockSpec((MM, HH), lambda e, i: (e * GG + i, 0)),
                  pl.BlockSpec((MM, HH), lambda e, i: (e * GG + i, 0)),
                  pl.BlockSpec((1, 1, HH), lambda e, i: (e, 0, 0)),
                  pl.BlockSpec((1, 1, 1, MM), lambda e, i: (e, i, 0, 0))],
        out_specs=[pl.BlockSpec((1, BB, HH), lambda e, i: (e, 0, 0)),
                   pl.BlockSpec((1, 1, BB), lambda e, i: (e, 0, 0))],
        out_shape=[jax.ShapeDtypeStruct((NC, BB, HH), jnp.float32),
                   jax.ShapeDtypeStruct((NC, 1, BB), jnp.float32)],
    )(agg, rt2, b2b, batchb)


def _tail_body(ps_ref, pc_ref, d_ref, w1a_ref, w1b_ref,
               w1c_ref, b1_ref, w2r_ref, b2_ref, out_ref):
    se = ps_ref[0] / jnp.maximum(pc_ref[0], 1.0)
    ge = ps_ref[1] / jnp.maximum(pc_ref[1], 1.0)
    d = d_ref[...]
    dm = jnp.mean(d)
    sd = jnp.sqrt(jnp.mean((d - dm) ** 2))
    dn = (d - dm) / (sd + 1e-6)
    z = (jnp.dot(se, w1a_ref[...], preferred_element_type=jnp.float32)
         + jnp.dot(ge, w1b_ref[...], preferred_element_type=jnp.float32)
         + dn * w1c_ref[...] + b1_ref[...])
    hh = jnp.maximum(z, 0.0)
    out_ref[...] = jnp.dot(hh, w2r_ref[...],
                           preferred_element_type=jnp.float32) + b2_ref[...]


def _tc_tail(ps, pc, d, w1a, w1b, w1c, b1, w2r, b2):
    return pl.pallas_call(
        _tail_body,
        out_shape=jax.ShapeDtypeStruct((BB, 1), jnp.float32),
    )(ps, pc, d, w1a, w1b, w1c, b1, w2r, b2)


def kernel(state_x, state_edge_index, state_edge_type, state_batch,
           goal_x, goal_edge_index, goal_edge_type, goal_batch, depth,
           s1_W, s1_root, s1_b, s2_W, s2_root, s2_b,
           g1_W, g1_root, g1_b, g2_W, g2_root, g2_b,
           reg_W1, reg_b1, reg_W2, reg_b2):
    srcb = jnp.stack([state_edge_index[0], goal_edge_index[0]])
    dstb = jnp.stack([state_edge_index[1], goal_edge_index[1]])
    etb = jnp.stack([state_edge_type, goal_edge_type])
    xb = jnp.stack([state_x, goal_x])
    batchb = jnp.stack([state_batch, goal_batch]).reshape(NC, GG, 1, MM)

    def wr(W):
        return W.transpose(1, 0, 2).reshape(W.shape[1], RR * HH)

    xrel1, rt1 = _tc_mm1(xb, jnp.stack([wr(s1_W), wr(g1_W)]),
                         jnp.stack([s1_root, g1_root]))
    normb, agg1 = _sc_l1(srcb, dstb, etb, xrel1.reshape(NC * NRR, HH))
    xrel2, rt2 = _tc_mm2(agg1, rt1,
                         jnp.stack([s1_b, g1_b]).reshape(NC, 1, HH),
                         jnp.stack([wr(s2_W), wr(g2_W)]),
                         jnp.stack([s2_root, g2_root]))
    agg2 = _sc_l2(srcb, dstb, etb, normb, xrel2.reshape(NC * NRR, HH))
    ps, pc = _tc_pool(agg2, rt2,
                      jnp.stack([s2_b, g2_b]).reshape(NC, 1, HH), batchb)
    pred = _tc_tail(ps, jnp.swapaxes(pc, 1, 2),
                    depth.reshape(BB, 1),
                    reg_W1[:HH], reg_W1[HH:2 * HH], reg_W1[2 * HH:],
                    reg_b1.reshape(1, HH), reg_W2,
                    reg_b2.reshape(1, 1))
    return pred.reshape(BB)
